# baseline (speedup 1.0000x reference)
import functools
import math

import jax
import jax.numpy as jnp
from jax import lax
from jax.experimental import pallas as pl
from jax.experimental.pallas import tpu as pltpu
from jax.experimental.pallas import tpu_sc as plsc

F32 = jnp.float32
I32 = jnp.int32

DEPTH = 3
N_BLOCKS = 2
RATIO = 0.5
HID = 128
LAT = 64
TB = 256
NCHS = (98, 40, 20, 12)


def _round_up(a, m):
    return (a + m - 1) // m * m


@functools.lru_cache(maxsize=None)
def _make_segsum(S, npg, NCH, HR):
    mesh = plsc.VectorSubcoreMesh(core_axis_name="c", subcore_axis_name="s")
    rpt = S // 16

    @functools.partial(
        pl.kernel,
        mesh=mesh,
        out_type=jax.ShapeDtypeStruct((HR, HID), F32),
        scratch_types=[
            pltpu.VMEM((NCH, 128), I32),
            pltpu.VMEM((NCH, 128), I32),
            pltpu.VMEM((128, HID), F32),
            pltpu.VMEM_SHARED((S, HID), F32),
            pltpu.SemaphoreType.DMA,
        ],
    )
    def seg(h_hbm, src_hbm, dst_hbm, z_hbm, out_hbm, src_v, dst_v, rows_v,
            acc_sp, sem):
        c = lax.axis_index("c")
        s = lax.axis_index("s")
        tid = c * 16 + s
        pltpu.sync_copy(z_hbm.at[pl.ds(0, rpt)], acc_sp.at[pl.ds(s * rpt, rpt)])
        pltpu.sync_copy(src_hbm.at[tid], src_v)
        pltpu.sync_copy(dst_hbm.at[tid], dst_v)
        plsc.subcore_barrier()

        def body(b, carry):
            pltpu.async_copy(h_hbm.at[src_v.at[b]], rows_v, sem).wait()
            pltpu.sync_copy(rows_v, acc_sp.at[dst_v.at[b]], add=True)
            return carry

        lax.fori_loop(0, NCH, body, 0)
        plsc.subcore_barrier()
        pltpu.sync_copy(acc_sp.at[pl.ds(s * rpt, rpt)],
                        out_hbm.at[pl.ds(c * S + s * rpt, rpt)])

    return seg


def _silu(x):
    return x * jax.lax.logistic(x)


@functools.lru_cache(maxsize=None)
def _make_embed(S, npg, HR):
    def body(x_ref, pos_ref, wi_ref, bi_ref, w1_ref, b1_ref, w2_ref, b2_ref,
             o_ref):
        i = pl.program_id(0)
        h = jnp.dot(x_ref[...], wi_ref[...], preferred_element_type=F32)
        pe = _silu(jnp.dot(pos_ref[...], w1_ref[...],
                           preferred_element_type=F32) + b1_ref[...])
        pe = jnp.dot(pe, w2_ref[...], preferred_element_type=F32) + b2_ref[...]
        h = h + bi_ref[...] + pe
        q = i * TB + lax.broadcasted_iota(I32, (TB, 1), 0)
        r = jnp.where(q >= S, q - S, q)
        o_ref[...] = jnp.where((r < npg) & (q < 2 * S), h, 0.0)

    grid = HR // TB
    row = pl.BlockSpec((TB, HID), lambda i: (i, 0))
    full = pl.BlockSpec((HID, HID), lambda i: (0, 0))
    vec = pl.BlockSpec((1, HID), lambda i: (0, 0))
    return pl.pallas_call(
        body,
        grid=(grid,),
        in_specs=[row, row, full, vec, full, vec, full, vec],
        out_specs=row,
        out_shape=jax.ShapeDtypeStruct((HR, HID), F32),
    )


@functools.lru_cache(maxsize=None)
def _make_block(HR):
    def body(h_ref, a_ref, w_ref, b_ref, o_ref):
        u = _silu(jnp.dot(a_ref[...], w_ref[...],
                          preferred_element_type=F32) + b_ref[...])
        o_ref[...] = h_ref[...] + u

    grid = HR // TB
    row = pl.BlockSpec((TB, HID), lambda i: (i, 0))
    full = pl.BlockSpec((HID, HID), lambda i: (0, 0))
    vec = pl.BlockSpec((1, HID), lambda i: (0, 0))
    return pl.pallas_call(
        body,
        grid=(grid,),
        in_specs=[row, row, full, vec],
        out_specs=row,
        out_shape=jax.ShapeDtypeStruct((HR, HID), F32),
    )


@functools.lru_cache(maxsize=None)
def _make_head(S, npg, HR):
    def body(h_ref, mw_ref, mb_ref, lw_ref, lb_ref, mu_ref, lv_ref):
        h = h_ref[...]
        q = lax.broadcasted_iota(I32, (HR, 1), 0)
        inv = 1.0 / npg
        m0 = jnp.sum(jnp.where(q < npg, h, 0.0), axis=0, keepdims=True) * inv
        m1 = jnp.sum(jnp.where((q >= S) & (q < S + npg), h, 0.0),
                     axis=0, keepdims=True) * inv
        hagg = jnp.concatenate([m0, m1], axis=0)
        mu = jnp.dot(hagg, mw_ref[...], preferred_element_type=F32) + mb_ref[...]
        lv = jnp.dot(hagg, lw_ref[...], preferred_element_type=F32) + lb_ref[...]
        mu_ref[...] = mu
        lv_ref[...] = jnp.clip(lv, -10.0, 2.0)

    return pl.pallas_call(
        body,
        in_specs=[pl.BlockSpec(memory_space=pltpu.VMEM)] * 5,
        out_specs=[pl.BlockSpec(memory_space=pltpu.VMEM)] * 2,
        out_shape=[jax.ShapeDtypeStruct((2, LAT), F32),
                   jax.ShapeDtypeStruct((2, LAT), F32)],
    )


def _tile_edges(sg, dl, alive, g_of_e, S, npg, NCH, HR):
    E2 = sg.shape[0]
    rpt = S // 16
    cap = NCH * 128
    tile = jnp.where(alive, g_of_e * 16 + dl // rpt, 32).astype(I32)
    key = tile * S + jnp.where(alive, dl, 0)
    order = jnp.argsort(key)
    tile_s = tile[order]
    counts = jnp.bincount(tile, length=33)
    starts = jnp.concatenate([jnp.zeros((1,), counts.dtype),
                              jnp.cumsum(counts)[:-1]])
    slot = jnp.arange(E2, dtype=I32) - starts[tile_s].astype(I32)
    ok = (tile_s < 32) & (slot < cap)
    flat = jnp.where(ok, tile_s * cap + slot, 32 * cap)
    npad = jnp.arange(32 * cap + 1, dtype=I32)
    buf_s = (2 * S + npad % (HR - 2 * S)).at[flat].set(
        jnp.where(ok, sg[order], 2 * S))
    buf_d = (npg + npad % (S - npg)).at[flat].set(
        jnp.where(ok, dl[order], npg))
    return (buf_s[:32 * cap].reshape(32, NCH, 128),
            buf_d[:32 * cap].reshape(32, NCH, 128))


def _ln(h, g, b, valid):
    m = h.mean(axis=-1, keepdims=True)
    v = jnp.var(h, axis=-1, keepdims=True)
    out = (h - m) / jnp.sqrt(v + 1e-5) * g + b
    return jnp.where(valid, out, 0.0)


def kernel(x, edge_index, pos, batch_size, in_proj_W, in_proj_b, pos_W1,
           pos_b1, pos_W2, pos_b2, stage_W, stage_b, stage_gamma, stage_beta,
           pool_p, mu_W, mu_b, lv_W, lv_b):
    N = pos.shape[0]
    total = x.shape[0]
    B = total // N
    assert B == 2, "kernel specialized for B == 2"
    E = edge_index.shape[1]
    E2 = 2 * E
    bs_scale = jnp.asarray(batch_size, F32) / jnp.float32(B)

    npgs, Ss, HRs = [], [], []
    npg = N
    for d in range(DEPTH + 1):
        S = _round_up(npg, 128)
        npgs.append(npg)
        Ss.append(S)
        HRs.append(2 * S + TB)
        npg = int(math.ceil(RATIO * npg))

    zeros = jnp.zeros((Ss[0] // 16, HID), F32)
    g_of_e = jnp.concatenate([jnp.zeros((E,), I32), jnp.ones((E,), I32)])

    S0, npg0, HR0 = Ss[0], npgs[0], HRs[0]
    xg = x.reshape(B, N, HID)
    x_pad = jnp.pad(xg, ((0, 0), (0, S0 - N), (0, 0))).reshape(B * S0, HID)
    x_pad = jnp.pad(x_pad, ((0, HR0 - B * S0), (0, 0)))
    posg = jnp.pad(pos, ((0, S0 - N), (0, 125)))
    pos_pad = jnp.concatenate([posg, posg], axis=0)
    pos_pad = jnp.pad(pos_pad, ((0, HR0 - B * S0), (0, 0)))
    W1p = jnp.pad(pos_W1, ((0, 125), (0, 0)))

    ei0 = edge_index[0].astype(I32)
    ei1 = edge_index[1].astype(I32)
    sg = jnp.concatenate([ei0, ei0 + S0])
    dl = jnp.concatenate([ei1, ei1])
    alive = jnp.ones((E2,), jnp.bool_)

    row = lambda a: a.reshape(1, -1)
    h = _make_embed(S0, npg0, HR0)(
        x_pad, pos_pad, in_proj_W, row(in_proj_b), W1p, row(pos_b1), pos_W2,
        row(pos_b2))

    keeps = []
    for d in range(DEPTH + 1):
        S, npg, HR = Ss[d], npgs[d], HRs[d]
        NCH = NCHS[d]
        zs = zeros[: S // 16]
        q = jnp.arange(HR, dtype=I32)[:, None]
        r_ = jnp.where(q >= S, q - S, q)
        valid = (r_ < npg) & (q < 2 * S)
        bsrc, bdst = _tile_edges(sg, dl, alive, g_of_e, S, npg, NCH, HR)
        for bi in range(N_BLOCKS):
            W = stage_W[d, bi]
            if d == 0:
                W = W * bs_scale
            agg = _make_segsum(S, npg, NCH, HR)(h, bsrc, bdst, zs)
            y = _make_block(HR)(h, agg, W, row(stage_b[d, bi]))
            h = _ln(y, stage_gamma[d, bi], stage_beta[d, bi], valid)
        if d == DEPTH:
            break
        k = npgs[d + 1]
        S_n, HR_n = Ss[d + 1], HRs[d + 1]
        p = pool_p[d]
        score = (h @ p) / (jnp.linalg.norm(p) + 1e-8)
        sc2 = score[: 2 * S].reshape(2, S)[:, :npg]
        topi = jax.lax.top_k(sc2, k)[1].astype(I32)
        keeps.append((topi + (jnp.arange(2, dtype=I32) * npg)[:, None])
                     .reshape(-1))
        keep_pad = (topi + (jnp.arange(2, dtype=I32) * S)[:, None]).reshape(-1)
        hs = h * jnp.tanh(score)[:, None]
        hk = hs[keep_pad].reshape(2, k, HID)
        h = jnp.pad(hk, ((0, 0), (0, S_n - k), (0, 0))).reshape(2 * S_n, HID)
        h = jnp.pad(h, ((0, HR_n - 2 * S_n), (0, 0)))
        nid = jnp.full((2, S + 1), -1, I32)
        nid = nid.at[jnp.arange(2, dtype=I32)[:, None], topi].set(
            (jnp.arange(k, dtype=I32)[None, :]
             + jnp.arange(2, dtype=I32)[:, None] * S_n))
        sl = jnp.minimum(sg - g_of_e * S, S)
        snew = nid[g_of_e, sl]
        dnew = nid[g_of_e, jnp.minimum(dl, S)]
        alive = alive & (snew >= 0) & (dnew >= 0)
        sg = jnp.where(alive, snew, 0)
        dl = jnp.where(alive, dnew - g_of_e * S_n, 0)

    S3, npg3, HR3 = Ss[DEPTH], npgs[DEPTH], HRs[DEPTH]
    mu, logvar = _make_head(S3, npg3, HR3)(h, mu_W, row(mu_b), lv_W,
                                           row(lv_b))
    z = mu
    return (z, mu, logvar, keeps[0], keeps[1], keeps[2])
